# Initial kernel scaffold; baseline (speedup 1.0000x reference)
import jax
import jax.numpy as jnp
from jax.experimental import pallas as pl


def kernel(x, edge_index, edge_weight, W1, b1, W2, b2, W3, b3, fcW, fcb):
    raise NotImplementedError("write your pallas kernel here")


# speedup vs baseline: 1.0916x; 1.0916x over previous
import functools

import jax
import jax.numpy as jnp
from jax.experimental import pallas as pl
from jax.experimental.pallas import tpu as pltpu

N_NODES = 10000
D = 128


def _linear_kernel(x_ref, w_ref, o_ref):
    o_ref[...] = jnp.dot(x_ref[...], w_ref[...],
                         preferred_element_type=jnp.float32)


def _linear(x, w):
    return pl.pallas_call(
        _linear_kernel,
        out_shape=jax.ShapeDtypeStruct((x.shape[0], w.shape[1]), jnp.float32),
    )(x, w)


def _head_kernel(h_ref, fcw_ref, fcb_ref, y_ref):
    y = jnp.dot(h_ref[...], fcw_ref[...], preferred_element_type=jnp.float32)
    y = y + fcb_ref[...]
    m = jnp.max(y, axis=1, keepdims=True)
    e = jnp.exp(y - m)
    lse = jnp.log(jnp.sum(e, axis=1, keepdims=True)) + m
    y_ref[...] = y - lse


def _head(h, fcW, fcb):
    return pl.pallas_call(
        _head_kernel,
        out_shape=jax.ShapeDtypeStruct((h.shape[0], fcW.shape[1]), jnp.float32),
    )(h, fcW, fcb.reshape(1, -1))


def kernel(x, edge_index, edge_weight, W1, b1, W2, b2, W3, b3, fcW, fcb):
    src = edge_index[0]
    dst = edge_index[1]

    def conv(h, W, b):
        t = _linear(h, W)
        msg = edge_weight[:, None] * jnp.take(t, src, axis=0)
        agg = jax.ops.segment_sum(msg, dst, num_segments=N_NODES)
        return agg + b

    h = jax.nn.relu(conv(x, W1, b1))
    h = jax.nn.relu(conv(h, W2, b2))
    h = conv(h, W3, b3)
    y = _head(h, fcW, fcb)
    return h, y


# speedup vs baseline: 4.3795x; 4.0118x over previous
import dataclasses
import functools

import jax
import jax.numpy as jnp
from jax import lax
from jax.experimental import pallas as pl
from jax.experimental.pallas import tpu as pltpu
from jax.experimental.pallas import tpu_sc as plsc

N_NODES = 10000
N_EDGES = 320000
D = 128
G = 128
N_GROUPS = N_EDGES // G
NW = 32
ROWS_PER_SUB = 624
ROWS_TAIL = N_NODES - 16 * ROWS_PER_SUB
GROUPS_BASE = N_GROUPS // NW
GROUPS_EXTRA = N_GROUPS - GROUPS_BASE * NW

_f32 = jnp.float32


def _sc_agg_kernel(t_hbm, src_hbm, dst_hbm, norm_hbm, zeros_hbm, out_hbm,
                   idx_v, dst_v, norm_v, rows_v, agg_sp, sem):
    c = lax.axis_index("c")
    s = lax.axis_index("s")
    wid = s * 2 + c

    r0 = s * ROWS_PER_SUB
    pltpu.sync_copy(zeros_hbm.at[pl.ds(r0, ROWS_PER_SUB)],
                    agg_sp.at[pl.ds(r0, ROWS_PER_SUB)])

    @pl.when(s == 15)
    def _():
        pltpu.sync_copy(zeros_hbm.at[pl.ds(16 * ROWS_PER_SUB, ROWS_TAIL)],
                        agg_sp.at[pl.ds(16 * ROWS_PER_SUB, ROWS_TAIL)])

    plsc.subcore_barrier()

    base = wid * GROUPS_BASE + jnp.minimum(wid, GROUPS_EXTRA)
    cnt = GROUPS_BASE + (wid < GROUPS_EXTRA).astype(jnp.int32)

    def group_body(m, carry):
        e0 = (base + m) * G
        pltpu.sync_copy(src_hbm.at[pl.ds(e0, G)], idx_v)
        pltpu.sync_copy(dst_hbm.at[pl.ds(e0, G)], dst_v)
        pltpu.sync_copy(norm_hbm.at[pl.ds(e0, G)], norm_v)
        pltpu.async_copy(t_hbm.at[idx_v], rows_v, sem).wait()

        def scale_body(r, carry2):
            nb = plsc.load_gather(norm_v, [lax.broadcast(r, (16,))])
            for j in range(8):
                sl = pl.ds(j * 16, 16)
                rows_v[r, sl] = rows_v[r, sl] * nb
            return carry2

        lax.fori_loop(0, G, scale_body, 0, unroll=False)
        pltpu.sync_copy(rows_v, agg_sp.at[dst_v], add=True)
        return carry

    lax.fori_loop(0, cnt, group_body, 0)

    plsc.subcore_barrier()
    pltpu.sync_copy(agg_sp.at[pl.ds(r0, ROWS_PER_SUB)],
                    out_hbm.at[c].at[pl.ds(r0, ROWS_PER_SUB)])

    @pl.when(s == 15)
    def _():
        pltpu.sync_copy(agg_sp.at[pl.ds(16 * ROWS_PER_SUB, ROWS_TAIL)],
                        out_hbm.at[c].at[pl.ds(16 * ROWS_PER_SUB, ROWS_TAIL)])


@jax.jit
def _sc_aggregate(t, src2, dst2, norm2, zeros):
    mesh = plsc.VectorSubcoreMesh(core_axis_name="c", subcore_axis_name="s")
    cp = pltpu.CompilerParams()
    if "needs_layout_passes" in pltpu.CompilerParams.__dataclass_fields__:
        cp = dataclasses.replace(cp, needs_layout_passes=False)
    kfn = pl.kernel(
        _sc_agg_kernel,
        out_type=jax.ShapeDtypeStruct((2, N_NODES, D), _f32),
        mesh=mesh,
        scratch_types=[
            pltpu.VMEM((G,), jnp.int32),
            pltpu.VMEM((G,), jnp.int32),
            pltpu.VMEM((G,), _f32),
            pltpu.VMEM((G, D), _f32),
            pltpu.VMEM_SHARED((N_NODES, D), _f32),
            pltpu.SemaphoreType.DMA,
        ],
        compiler_params=cp,
    )
    return kfn(t, src2, dst2, norm2, zeros)


def _mm_first_kernel(x_ref, w_ref, o_ref):
    o_ref[...] = jnp.dot(x_ref[...], w_ref[...],
                         preferred_element_type=_f32)


def _mm_mid_kernel(p_ref, b_ref, w_ref, o_ref):
    h = jax.nn.relu(p_ref[0] + p_ref[1] + b_ref[...])
    o_ref[...] = jnp.dot(h, w_ref[...], preferred_element_type=_f32)


def _head_kernel(p_ref, b_ref, fcw_ref, fcb_ref, h_ref, y_ref):
    h = p_ref[0] + p_ref[1] + b_ref[...]
    h_ref[...] = h
    y = jnp.dot(h, fcw_ref[...], preferred_element_type=_f32)
    y = y + fcb_ref[...]
    m = jnp.max(y, axis=1, keepdims=True)
    e = jnp.exp(y - m)
    lse = jnp.log(jnp.sum(e, axis=1, keepdims=True)) + m
    y_ref[...] = y - lse


def _mm_first(x, w):
    return pl.pallas_call(
        _mm_first_kernel,
        out_shape=jax.ShapeDtypeStruct((x.shape[0], w.shape[1]), _f32),
    )(x, w)


def _mm_mid(p, b, w):
    return pl.pallas_call(
        _mm_mid_kernel,
        out_shape=jax.ShapeDtypeStruct((p.shape[1], w.shape[1]), _f32),
    )(p, b.reshape(1, -1), w)


def _head(p, b, fcW, fcb):
    return pl.pallas_call(
        _head_kernel,
        out_shape=(jax.ShapeDtypeStruct((p.shape[1], D), _f32),
                   jax.ShapeDtypeStruct((p.shape[1], fcW.shape[1]), _f32)),
    )(p, b.reshape(1, -1), fcW, fcb.reshape(1, -1))


def kernel(x, edge_index, edge_weight, W1, b1, W2, b2, W3, b3, fcW, fcb):
    src2 = edge_index[0]
    dst2 = edge_index[1]
    norm2 = edge_weight
    zeros = jnp.zeros((N_NODES, D), _f32)

    t1 = _mm_first(x, W1)
    p1 = _sc_aggregate(t1, src2, dst2, norm2, zeros)
    t2 = _mm_mid(p1, b1, W2)
    p2 = _sc_aggregate(t2, src2, dst2, norm2, zeros)
    t3 = _mm_mid(p2, b2, W3)
    p3 = _sc_aggregate(t3, src2, dst2, norm2, zeros)
    h, y = _head(p3, b3, fcW, fcb)
    return h, y


# speedup vs baseline: 11.2177x; 2.5614x over previous
import dataclasses
import functools

import jax
import jax.numpy as jnp
from jax import lax
from jax.experimental import pallas as pl
from jax.experimental.pallas import tpu as pltpu
from jax.experimental.pallas import tpu_sc as plsc

N_NODES = 10000
N_EDGES = 320000
D = 128
G = 128
NW = 32
GROUPS_PER_W = 81
E_PAD = NW * GROUPS_PER_W * G
NBUF = 3
N_ITERS = GROUPS_PER_W // NBUF
ROWS_PER_SUB = 624
ROWS_TAIL = N_NODES - 16 * ROWS_PER_SUB

_f32 = jnp.float32


def _sc_agg_kernel(t_hbm, src_hbm, dst_hbm, norm_hbm, zeros_hbm, out_hbm,
                   src0, src1, src2, dst0, dst1, dst2, nrm0, nrm1, nrm2,
                   rows0, rows1, rows2, agg_sp,
                   gs0, gs1, gs2, ss0, ss1, ss2, ds0, ds1, ds2, isem):
    c = lax.axis_index("c")
    s = lax.axis_index("s")
    wid = s * 2 + c
    srcb = (src0, src1, src2)
    dstb = (dst0, dst1, dst2)
    nrmb = (nrm0, nrm1, nrm2)
    rows = (rows0, rows1, rows2)
    gsem = (gs0, gs1, gs2)
    ssem = (ss0, ss1, ss2)
    dsem = (ds0, ds1, ds2)

    e_base = wid * (GROUPS_PER_W * G)

    def src_sl(g):
        return src_hbm.at[pl.ds(e_base + g * G, G)]

    def dst_sl(g):
        return dst_hbm.at[pl.ds(e_base + g * G, G)]

    def nrm_sl(g):
        return norm_hbm.at[pl.ds(e_base + g * G, G)]

    for k in range(2):
        pltpu.sync_copy(src_sl(k), srcb[k])
        pltpu.sync_copy(nrm_sl(k), nrmb[k])
        pltpu.make_async_copy(dst_sl(k), dstb[k], dsem[k]).start()
        pltpu.make_async_copy(t_hbm.at[srcb[k]], rows[k], gsem[k]).start()

    r0 = s * ROWS_PER_SUB
    pltpu.sync_copy(zeros_hbm.at[pl.ds(r0, ROWS_PER_SUB)],
                    agg_sp.at[pl.ds(r0, ROWS_PER_SUB)])

    @pl.when(s == 15)
    def _():
        pltpu.sync_copy(zeros_hbm.at[pl.ds(16 * ROWS_PER_SUB, ROWS_TAIL)],
                        agg_sp.at[pl.ds(16 * ROWS_PER_SUB, ROWS_TAIL)])

    plsc.subcore_barrier()

    def _scale(buf, nbuf):
        def tile_body(r16, carry):
            for dr in range(4):
                r = r16 + dr
                nb = plsc.load_gather(nbuf, [lax.broadcast(r, (16,))])
                for j in range(8):
                    sl = pl.ds(j * 16, 16)
                    buf[r, sl] = buf[r, sl] * nb
            return carry

        lax.fori_loop(0, G // 4, lambda t, cy: tile_body(t * 4, cy), 0)

    def iter_body(i, carry):
        for k in range(NBUF):
            g = i * NBUF + k
            bp = (k + 2) % NBUF
            gp2 = g + 2
            have_next = gp2 <= GROUPS_PER_W - 1

            @pl.when(have_next)
            def _():
                pltpu.make_async_copy(src_sl(gp2), srcb[bp], isem).start()
                pltpu.make_async_copy(nrm_sl(gp2), nrmb[bp], isem).start()

            pltpu.make_async_copy(t_hbm.at[srcb[k]], rows[k],
                                  gsem[k]).wait()
            _scale(rows[k], nrmb[k])
            pltpu.make_async_copy(dst_sl(g), dstb[k], dsem[k]).wait()
            pltpu.make_async_copy(rows[k], agg_sp.at[dstb[k]],
                                  ssem[k]).start(add=True)

            @pl.when(g >= 1)
            def _():
                pltpu.make_async_copy(rows[bp], agg_sp.at[dstb[bp]],
                                      ssem[bp]).wait()

            @pl.when(have_next)
            def _():
                pltpu.make_async_copy(dst_sl(gp2), dstb[bp],
                                      dsem[bp]).start()
                pltpu.make_async_copy(src_sl(gp2), srcb[bp], isem).wait()
                pltpu.make_async_copy(nrm_sl(gp2), nrmb[bp], isem).wait()
                pltpu.make_async_copy(t_hbm.at[srcb[bp]], rows[bp],
                                      gsem[bp]).start()
        return carry

    lax.fori_loop(0, N_ITERS, iter_body, 0)

    kl = (GROUPS_PER_W - 1) % NBUF
    pltpu.make_async_copy(rows[kl], agg_sp.at[dstb[kl]], ssem[kl]).wait()

    plsc.subcore_barrier()
    pltpu.sync_copy(agg_sp.at[pl.ds(r0, ROWS_PER_SUB)],
                    out_hbm.at[c].at[pl.ds(r0, ROWS_PER_SUB)])

    @pl.when(s == 15)
    def _():
        pltpu.sync_copy(agg_sp.at[pl.ds(16 * ROWS_PER_SUB, ROWS_TAIL)],
                        out_hbm.at[c].at[pl.ds(16 * ROWS_PER_SUB, ROWS_TAIL)])


@jax.jit
def _sc_aggregate(t, src_e, dst_e, norm_e, zeros):
    mesh = plsc.VectorSubcoreMesh(core_axis_name="c", subcore_axis_name="s")
    cp = pltpu.CompilerParams()
    if "needs_layout_passes" in pltpu.CompilerParams.__dataclass_fields__:
        cp = dataclasses.replace(cp, needs_layout_passes=False)
    kfn = pl.kernel(
        _sc_agg_kernel,
        out_type=jax.ShapeDtypeStruct((2, N_NODES, D), _f32),
        mesh=mesh,
        scratch_types=(
            [pltpu.VMEM((G,), jnp.int32) for _ in range(3)]
            + [pltpu.VMEM((G,), jnp.int32) for _ in range(3)]
            + [pltpu.VMEM((G,), _f32) for _ in range(3)]
            + [pltpu.VMEM((G, D), _f32) for _ in range(3)]
            + [pltpu.VMEM_SHARED((N_NODES, D), _f32)]
            + [pltpu.SemaphoreType.DMA for _ in range(10)]
        ),
        compiler_params=cp,
    )
    return kfn(t, src_e, dst_e, norm_e, zeros)


def _mm_first_kernel(x_ref, w_ref, o_ref):
    o_ref[...] = jnp.dot(x_ref[...], w_ref[...],
                         preferred_element_type=_f32)


def _mm_mid_kernel(p_ref, b_ref, w_ref, o_ref):
    h = jax.nn.relu(p_ref[0] + p_ref[1] + b_ref[...])
    o_ref[...] = jnp.dot(h, w_ref[...], preferred_element_type=_f32)


def _head_kernel(p_ref, b_ref, fcw_ref, fcb_ref, h_ref, y_ref):
    h = p_ref[0] + p_ref[1] + b_ref[...]
    h_ref[...] = h
    y = jnp.dot(h, fcw_ref[...], preferred_element_type=_f32)
    y = y + fcb_ref[...]
    m = jnp.max(y, axis=1, keepdims=True)
    e = jnp.exp(y - m)
    lse = jnp.log(jnp.sum(e, axis=1, keepdims=True)) + m
    y_ref[...] = y - lse


def _mm_first(x, w):
    return pl.pallas_call(
        _mm_first_kernel,
        out_shape=jax.ShapeDtypeStruct((x.shape[0], w.shape[1]), _f32),
    )(x, w)


def _mm_mid(p, b, w):
    return pl.pallas_call(
        _mm_mid_kernel,
        out_shape=jax.ShapeDtypeStruct((p.shape[1], w.shape[1]), _f32),
    )(p, b.reshape(1, -1), w)


def _head(p, b, fcW, fcb):
    return pl.pallas_call(
        _head_kernel,
        out_shape=(jax.ShapeDtypeStruct((p.shape[1], D), _f32),
                   jax.ShapeDtypeStruct((p.shape[1], fcW.shape[1]), _f32)),
    )(p, b.reshape(1, -1), fcW, fcb.reshape(1, -1))


def kernel(x, edge_index, edge_weight, W1, b1, W2, b2, W3, b3, fcW, fcb):
    pad = E_PAD - N_EDGES
    pad_idx = (jnp.arange(pad, dtype=jnp.int32) * 13) % N_NODES
    src_e = jnp.concatenate([edge_index[0], pad_idx])
    dst_e = jnp.concatenate([edge_index[1], pad_idx])
    norm_e = jnp.concatenate([edge_weight, jnp.zeros((pad,), _f32)])
    zeros = jnp.zeros((N_NODES, D), _f32)

    t1 = _mm_first(x, W1)
    p1 = _sc_aggregate(t1, src_e, dst_e, norm_e, zeros)
    t2 = _mm_mid(p1, b1, W2)
    p2 = _sc_aggregate(t2, src_e, dst_e, norm_e, zeros)
    t3 = _mm_mid(p2, b2, W3)
    p3 = _sc_aggregate(t3, src_e, dst_e, norm_e, zeros)
    h, y = _head(p3, b3, fcW, fcb)
    return h, y
